# Initial kernel scaffold; baseline (speedup 1.0000x reference)
import jax
import jax.numpy as jnp
from jax.experimental import pallas as pl


def kernel(feature, edge_index, W, b):
    raise NotImplementedError("write your pallas kernel here")


# speedup vs baseline: 5.5113x; 5.5113x over previous
import functools

import jax
import jax.numpy as jnp
from jax import lax
from jax.experimental import pallas as pl
from jax.experimental.pallas import tpu as pltpu
from jax.experimental.pallas import tpu_sc as plsc

N = 10000
E = 320000
D = 128

NC = 2
NS = 16
NW = NC * NS
EPW = E // NW
CHUNK = 80
NCHUNK = EPW // CHUNK
NP = 10240
ROWS_PER_TILE = NP // NS
ZROWS = 128


def _sc_partials(feature, src, dst):
    mesh = plsc.VectorSubcoreMesh(core_axis_name="c", subcore_axis_name="s")

    @functools.partial(
        pl.kernel,
        mesh=mesh,
        out_type=jax.ShapeDtypeStruct((NC, NP, D), jnp.float32),
        scratch_types=[
            pltpu.VMEM((CHUNK,), jnp.int32),
            pltpu.VMEM((CHUNK,), jnp.int32),
            pltpu.VMEM((CHUNK, D), jnp.float32),
            pltpu.VMEM((ZROWS, D), jnp.float32),
            pltpu.VMEM_SHARED((NP, D), jnp.float32),
            pltpu.SemaphoreType.DMA,
        ],
    )
    def k(feat_hbm, src_hbm, dst_hbm, out_hbm, sidx, didx, rows, zbuf, acc, sem):
        cid = lax.axis_index("c")
        sid = lax.axis_index("s")
        wid = sid * NC + cid

        def zrow(r, carry):
            def zcol(j, carry2):
                zbuf[r, pl.ds(j * 16, 16)] = jnp.zeros((16,), jnp.float32)
                return carry2
            return lax.fori_loop(0, D // 16, zcol, carry)
        lax.fori_loop(0, ZROWS, zrow, 0)
        for kk in range(ROWS_PER_TILE // ZROWS):
            pltpu.sync_copy(zbuf, acc.at[pl.ds(sid * ROWS_PER_TILE + kk * ZROWS, ZROWS)])
        plsc.subcore_barrier()

        def body(i, carry):
            base = wid * EPW + i * CHUNK
            pltpu.sync_copy(src_hbm.at[pl.ds(base, CHUNK)], sidx)
            pltpu.sync_copy(dst_hbm.at[pl.ds(base, CHUNK)], didx)
            pltpu.async_copy(feat_hbm.at[sidx], rows, sem).wait()
            pltpu.sync_copy(rows, acc.at[didx], add=True)
            return carry
        lax.fori_loop(0, NCHUNK, body, 0)
        plsc.subcore_barrier()

        for kk in range(ROWS_PER_TILE // ZROWS):
            r0 = sid * ROWS_PER_TILE + kk * ZROWS
            pltpu.sync_copy(acc.at[pl.ds(r0, ZROWS)], out_hbm.at[cid, pl.ds(r0, ZROWS)])

    return k(feature, src, dst)


BLK = 1000


def _tc_linear_kernel(p_ref, w_ref, b_ref, out_ref):
    x = p_ref[0] + p_ref[1]
    y = lax.dot_general(
        x, w_ref[...], (((1,), (1,)), ((), ())),
        preferred_element_type=jnp.float32,
        precision=lax.Precision.HIGHEST,
    )
    out_ref[...] = y + b_ref[...]


def _tc_linear(partials, W, b):
    return pl.pallas_call(
        _tc_linear_kernel,
        grid=(N // BLK,),
        in_specs=[
            pl.BlockSpec((NC, BLK, D), lambda i: (0, i, 0)),
            pl.BlockSpec((D, D), lambda i: (0, 0)),
            pl.BlockSpec((1, D), lambda i: (0, 0)),
        ],
        out_specs=pl.BlockSpec((BLK, D), lambda i: (i, 0)),
        out_shape=jax.ShapeDtypeStruct((N, D), jnp.float32),
    )(partials, W, b.reshape(1, D))


@jax.jit
def kernel(feature, edge_index, W, b):
    src = edge_index[0]
    dst = edge_index[1]
    partials = _sc_partials(feature, src, dst)
    return _tc_linear(partials, W, b)


# speedup vs baseline: 10.4619x; 1.8982x over previous
import functools

import jax
import jax.numpy as jnp
from jax import lax
from jax.experimental import pallas as pl
from jax.experimental.pallas import tpu as pltpu
from jax.experimental.pallas import tpu_sc as plsc

N = 10000
E = 320000
D = 128

NC = 2
NS = 16
NW = NC * NS
EPW = E // NW
CHUNK = 100
NCHUNK = EPW // CHUNK
NSUP = NCHUNK // 2
NP = 10240
ROWS_PER_TILE = NP // NS
ZROWS = 64


def _sc_partials(feature, src2, dst2):
    mesh = plsc.VectorSubcoreMesh(core_axis_name="c", subcore_axis_name="s")

    @functools.partial(
        pl.kernel,
        mesh=mesh,
        out_type=jax.ShapeDtypeStruct((NC, NP, D), jnp.float32),
        scratch_types=[
            pltpu.VMEM((NCHUNK, CHUNK), jnp.int32),
            pltpu.VMEM((2, CHUNK), jnp.int32),
            pltpu.VMEM((2, CHUNK, D), jnp.float32),
            pltpu.VMEM((ZROWS, D), jnp.float32),
            pltpu.VMEM_SHARED((NP, D), jnp.float32),
            pltpu.SemaphoreType.DMA,
            pltpu.SemaphoreType.DMA,
            pltpu.SemaphoreType.DMA,
            pltpu.SemaphoreType.DMA,
            pltpu.SemaphoreType.DMA,
        ],
    )
    def k(feat_hbm, src_hbm, dst_hbm, out_hbm, sidx, didx, rows, zbuf, acc,
          isem, gsem0, gsem1, dsem0, dsem1):
        cid = lax.axis_index("c")
        sid = lax.axis_index("s")
        wid = sid * NC + cid

        ic0 = pltpu.async_copy(src_hbm.at[wid], sidx, isem)

        def zrow(r, carry):
            def zcol(j, carry2):
                zbuf[r, pl.ds(j * 16, 16)] = jnp.zeros((16,), jnp.float32)
                return carry2
            return lax.fori_loop(0, D // 16, zcol, carry)
        lax.fori_loop(0, ZROWS, zrow, 0)
        for kk in range(ROWS_PER_TILE // ZROWS):
            pltpu.sync_copy(zbuf, acc.at[pl.ds(sid * ROWS_PER_TILE + kk * ZROWS, ZROWS)])
        ic0.wait()
        plsc.subcore_barrier()

        gsems = (gsem0, gsem1)
        dsems = (dsem0, dsem1)

        def issue_gather(i, b):
            pltpu.async_copy(feat_hbm.at[sidx.at[i]], rows.at[b], gsems[b])
            pltpu.async_copy(dst_hbm.at[wid, i], didx.at[b], dsems[b])

        def wait_gather(i, b):
            pltpu.make_async_copy(feat_hbm.at[sidx.at[i]], rows.at[b], gsems[b]).wait()
            pltpu.make_async_copy(dst_hbm.at[wid, i], didx.at[b], dsems[b]).wait()

        def scatter(i, b):
            pltpu.sync_copy(rows.at[b], acc.at[didx.at[b]], add=True)

        issue_gather(0, 0)

        def body(j, carry):
            a = 2 * j
            wait_gather(a, 0)
            issue_gather(a + 1, 1)
            scatter(a, 0)
            wait_gather(a + 1, 1)
            issue_gather(a + 2, 0)
            scatter(a + 1, 1)
            return carry
        lax.fori_loop(0, NSUP - 1, body, 0)

        a = 2 * (NSUP - 1)
        wait_gather(a, 0)
        issue_gather(a + 1, 1)
        scatter(a, 0)
        wait_gather(a + 1, 1)
        scatter(a + 1, 1)
        plsc.subcore_barrier()

        for kk in range(ROWS_PER_TILE // ZROWS):
            r0 = sid * ROWS_PER_TILE + kk * ZROWS
            pltpu.sync_copy(acc.at[pl.ds(r0, ZROWS)], out_hbm.at[cid, pl.ds(r0, ZROWS)])

    return k(feature, src2, dst2)


BLK = 1000


def _tc_linear_kernel(p_ref, w_ref, b_ref, out_ref):
    x = p_ref[0] + p_ref[1]
    y = lax.dot_general(
        x, w_ref[...], (((1,), (1,)), ((), ())),
        preferred_element_type=jnp.float32,
        precision=lax.Precision.HIGHEST,
    )
    out_ref[...] = y + b_ref[...]


def _tc_linear(partials, W, b):
    return pl.pallas_call(
        _tc_linear_kernel,
        grid=(N // BLK,),
        in_specs=[
            pl.BlockSpec((NC, BLK, D), lambda i: (0, i, 0)),
            pl.BlockSpec((D, D), lambda i: (0, 0)),
            pl.BlockSpec((1, D), lambda i: (0, 0)),
        ],
        out_specs=pl.BlockSpec((BLK, D), lambda i: (i, 0)),
        out_shape=jax.ShapeDtypeStruct((N, D), jnp.float32),
    )(partials, W, b.reshape(1, D))


@jax.jit
def kernel(feature, edge_index, W, b):
    src2 = edge_index[0].reshape(NW, NCHUNK, CHUNK)
    dst2 = edge_index[1].reshape(NW, NCHUNK, CHUNK)
    partials = _sc_partials(feature, src2, dst2)
    return _tc_linear(partials, W, b)


# speedup vs baseline: 13.9555x; 1.3339x over previous
import functools

import jax
import jax.numpy as jnp
from jax import lax
from jax.experimental import pallas as pl
from jax.experimental.pallas import tpu as pltpu
from jax.experimental.pallas import tpu_sc as plsc

N = 10000
E = 320000
D = 128

NC = 2
NS = 16
NW = NC * NS
EPW = E // NW
CHUNK = 100
NCHUNK = EPW // CHUNK
NB = 3
NR = 2 * NB
NP = 10240
ROWS_PER_TILE = NP // NS
ZCOPY = 80


def _sc_partials(feature, src2, dst2):
    mesh = plsc.VectorSubcoreMesh(core_axis_name="c", subcore_axis_name="s")

    @functools.partial(
        pl.kernel,
        mesh=mesh,
        out_type=jax.ShapeDtypeStruct((NC, NP, D), jnp.float32),
        scratch_types=[
            pltpu.VMEM((NR, CHUNK), jnp.int32),
            pltpu.VMEM((NR, CHUNK), jnp.int32),
            pltpu.VMEM((NB, CHUNK, D), jnp.float32),
            pltpu.VMEM_SHARED((NP, D), jnp.float32),
            [pltpu.SemaphoreType.DMA] * NR,
            [pltpu.SemaphoreType.DMA] * NB,
        ],
    )
    def k(feat_hbm, src_hbm, dst_hbm, out_hbm, sring, dring, rows, acc,
          isems, gsems):
        cid = lax.axis_index("c")
        sid = lax.axis_index("s")
        wid = sid * NC + cid

        def zrow(r, carry):
            def zcol(j, carry2):
                rows[0, r, pl.ds(j * 16, 16)] = jnp.zeros((16,), jnp.float32)
                return carry2
            return lax.fori_loop(0, D // 16, zcol, carry)
        lax.fori_loop(0, ZCOPY, zrow, 0)
        for kk in range(ROWS_PER_TILE // ZCOPY):
            pltpu.sync_copy(rows.at[0, pl.ds(0, ZCOPY)],
                            acc.at[pl.ds(sid * ROWS_PER_TILE + kk * ZCOPY, ZCOPY)])
        plsc.subcore_barrier()

        def load_idx(i, s):
            pltpu.async_copy(src_hbm.at[wid, i], sring.at[s], isems[s])
            pltpu.async_copy(dst_hbm.at[wid, i], dring.at[s], isems[s])

        def wait_idx(i, s):
            pltpu.make_async_copy(src_hbm.at[wid, i], sring.at[s], isems[s]).wait()
            pltpu.make_async_copy(dst_hbm.at[wid, i], dring.at[s], isems[s]).wait()

        def issue_g(b, s):
            pltpu.async_copy(feat_hbm.at[sring.at[s]], rows.at[b], gsems[b])

        def wait_g(b, s):
            pltpu.make_async_copy(feat_hbm.at[sring.at[s]], rows.at[b], gsems[b]).wait()

        def scatter(b, s):
            pltpu.sync_copy(rows.at[b], acc.at[dring.at[s]], add=True)

        for s in range(NR):
            load_idx(s, s)
        for b in range(NB):
            wait_idx(b, b)
            issue_g(b, b)

        NGRP = (NCHUNK - NR) // NR

        def body(j, carry):
            a = NR * j
            for u in range(NR):
                b = u % NB
                wait_g(b, u)
                scatter(b, u)
                load_idx(a + u + NR, u)
                wait_idx(a + u + NB, (u + NB) % NR)
                issue_g(b, (u + NB) % NR)
            return carry
        lax.fori_loop(0, NGRP, body, 0)

        for i in range(NGRP * NR, NCHUNK):
            b = i % NB
            u = i % NR
            wait_g(b, u)
            scatter(b, u)
            if i + NR < NCHUNK:
                load_idx(i + NR, u)
            if i + NB < NCHUNK:
                wait_idx(i + NB, (i + NB) % NR)
                issue_g(b, (i + NB) % NR)
        plsc.subcore_barrier()

        for kk in range(ROWS_PER_TILE // ZCOPY):
            r0 = sid * ROWS_PER_TILE + kk * ZCOPY
            pltpu.sync_copy(acc.at[pl.ds(r0, ZCOPY)], out_hbm.at[cid, pl.ds(r0, ZCOPY)])

    return k(feature, src2, dst2)


BLK = 1000


def _tc_linear_kernel(p_ref, w_ref, b_ref, out_ref):
    x = p_ref[0] + p_ref[1]
    y = lax.dot_general(
        x, w_ref[...], (((1,), (1,)), ((), ())),
        preferred_element_type=jnp.float32,
        precision=lax.Precision.HIGHEST,
    )
    out_ref[...] = y + b_ref[...]


def _tc_linear(partials, W, b):
    return pl.pallas_call(
        _tc_linear_kernel,
        grid=(N // BLK,),
        in_specs=[
            pl.BlockSpec((NC, BLK, D), lambda i: (0, i, 0)),
            pl.BlockSpec((D, D), lambda i: (0, 0)),
            pl.BlockSpec((1, D), lambda i: (0, 0)),
        ],
        out_specs=pl.BlockSpec((BLK, D), lambda i: (i, 0)),
        out_shape=jax.ShapeDtypeStruct((N, D), jnp.float32),
    )(partials, W, b.reshape(1, D))


@jax.jit
def kernel(feature, edge_index, W, b):
    src2 = edge_index[0].reshape(NW, NCHUNK, CHUNK)
    dst2 = edge_index[1].reshape(NW, NCHUNK, CHUNK)
    partials = _sc_partials(feature, src2, dst2)
    return _tc_linear(partials, W, b)
